# Initial kernel scaffold; baseline (speedup 1.0000x reference)
import jax
import jax.numpy as jnp
from jax.experimental import pallas as pl


def kernel(image, n):
    raise NotImplementedError("write your pallas kernel here")


# speedup vs baseline: 3.9867x; 3.9867x over previous
import jax
import jax.numpy as jnp
from jax import lax
from jax.experimental import pallas as pl
from jax.experimental.pallas import tpu as pltpu

_PH = _PW = _PD = 8
_N = 8192
_D = 512
_K = 8
_RB = 256
_NBLK = _N // _RB


def _extract_patches(image):
    B, C, H, W, Dd = image.shape
    nh, nw, nd = H // _PH, W // _PW, Dd // _PD
    x = image.reshape(B, C, nh, _PH, nw, _PW, nd, _PD)
    x = jnp.transpose(x, (0, 2, 4, 6, 1, 3, 5, 7))
    return x.reshape(B, nh * nw * nd, C * _PH * _PW * _PD)


def _tc_body(feat_ref, featT_ref, idx_ref, sq_ref):
    i = pl.program_id(0)

    @pl.when(i == 0)
    def _():
        ft = featT_ref[...]
        sq_ref[...] = jnp.sum(ft * ft, axis=0, keepdims=True)

    score = sq_ref[...] - 2.0 * jnp.dot(
        feat_ref[...], featT_ref[...], preferred_element_type=jnp.float32
    )
    col = lax.broadcasted_iota(jnp.int32, (_RB, _N), 1)
    gid = lax.broadcasted_iota(jnp.int32, (_RB, 1), 0) + i * _RB
    score = jnp.where(col == gid, jnp.inf, score)

    for k in range(_K):
        m = jnp.min(score, axis=1, keepdims=True)
        a = jnp.min(jnp.where(score == m, col, jnp.int32(_N)),
                    axis=1, keepdims=True)
        score = jnp.where(col == a, jnp.inf, score)
        idx_ref[:, k:k + 1] = a


def _tc_knn(feat, featT):
    return pl.pallas_call(
        _tc_body,
        grid=(_NBLK,),
        in_specs=[
            pl.BlockSpec((_RB, _D), lambda i: (i, 0)),
            pl.BlockSpec((_D, _N), lambda i: (0, 0)),
        ],
        out_specs=pl.BlockSpec((_RB, _K), lambda i: (i, 0)),
        out_shape=jax.ShapeDtypeStruct((_N, _K), jnp.int32),
        scratch_shapes=[pltpu.VMEM((1, _N), jnp.float32)],
    )(feat, featT)


def _adj_body(vals_ref, idx_ref, idxT_ref, out_ref):
    i = pl.program_id(0)
    col = lax.broadcasted_iota(jnp.int32, (_RB, _N), 1)
    gid = lax.broadcasted_iota(jnp.int32, (_RB, 1), 0) + i * _RB
    acc = jnp.zeros((_RB, _N), jnp.float32)
    for k in range(_K):
        vk = vals_ref[k]
        acc = acc + jnp.where(col == idx_ref[:, k:k + 1], vk, 0.0)
        acc = acc + jnp.where(idxT_ref[k:k + 1, :] == gid, vk, 0.0)
    out_ref[...] = acc


def _adj_build(vals8, idx, idxT):
    return pl.pallas_call(
        _adj_body,
        grid=(_NBLK,),
        in_specs=[
            pl.BlockSpec(memory_space=pltpu.SMEM),
            pl.BlockSpec((_RB, _K), lambda i: (i, 0)),
            pl.BlockSpec((_K, _N), lambda i: (0, 0)),
        ],
        out_specs=pl.BlockSpec((_RB, _N), lambda i: (i, 0)),
        out_shape=jax.ShapeDtypeStruct((_N, _N), jnp.float32),
    )(vals8, idx, idxT)


def kernel(image, n=8):
    features = _extract_patches(image)
    feat = features[0]
    vals8 = jnp.where(jnp.arange(_K) < n, 0.5, 0.0).astype(jnp.float32)
    idx = _tc_knn(feat, feat.T)
    adjacency = _adj_build(vals8, idx, idx.T)[None]
    return features, adjacency


# speedup vs baseline: 5.2726x; 1.3226x over previous
import jax
import jax.numpy as jnp
from jax import lax
from jax.experimental import pallas as pl
from jax.experimental.pallas import tpu as pltpu

_PH = _PW = _PD = 8
_N = 8192
_D = 512
_K = 8
_RB = 256
_NBLK = _N // _RB


def _extract_patches(image):
    B, C, H, W, Dd = image.shape
    nh, nw, nd = H // _PH, W // _PW, Dd // _PD
    x = image.reshape(B, C, nh, _PH, nw, _PW, nd, _PD)
    x = jnp.transpose(x, (0, 2, 4, 6, 1, 3, 5, 7))
    return x.reshape(B, nh * nw * nd, C * _PH * _PW * _PD)


def _tc_body(feat_ref, featA_ref, idx_ref, sq_ref):
    i = pl.program_id(0)

    @pl.when(i == 0)
    def _():
        fa = featA_ref[...]
        sq_ref[...] = jnp.sum(fa * fa, axis=1)[None, :]

    score = sq_ref[...] - 2.0 * lax.dot_general(
        feat_ref[...], featA_ref[...],
        dimension_numbers=(((1,), (1,)), ((), ())),
        preferred_element_type=jnp.float32,
    )
    col = lax.broadcasted_iota(jnp.int32, (_RB, _N), 1).astype(jnp.float32)
    gid = (lax.broadcasted_iota(jnp.int32, (_RB, 1), 0) + i * _RB).astype(
        jnp.float32)
    score = jnp.where(col == gid, jnp.inf, score)

    for k in range(_K):
        m = jnp.min(score, axis=1, keepdims=True)
        a = jnp.min(jnp.where(score == m, col, jnp.float32(_N)),
                    axis=1, keepdims=True)
        score = jnp.where(col == a, jnp.inf, score)
        idx_ref[:, k:k + 1] = a.astype(jnp.int32)


def _tc_knn(feat):
    return pl.pallas_call(
        _tc_body,
        grid=(_NBLK,),
        in_specs=[
            pl.BlockSpec((_RB, _D), lambda i: (i, 0)),
            pl.BlockSpec((_N, _D), lambda i: (0, 0)),
        ],
        out_specs=pl.BlockSpec((_RB, _K), lambda i: (i, 0)),
        out_shape=jax.ShapeDtypeStruct((_N, _K), jnp.int32),
        scratch_shapes=[pltpu.VMEM((1, _N), jnp.float32)],
    )(feat, feat)


def _adj_body(vals_ref, idx_ref, idxT_ref, out_ref):
    i = pl.program_id(0)
    col = lax.broadcasted_iota(jnp.int16, (_RB, _N), 1)
    gid = lax.broadcasted_iota(jnp.int16, (_RB, 1), 0) + (i * _RB).astype(jnp.int16)
    idxT = idxT_ref[...]
    acc = jnp.zeros((_RB, _N), jnp.int16)
    for k in range(_K):
        vk = vals_ref[k].astype(jnp.int16)
        idxk = idx_ref[:, k:k + 1].astype(jnp.int16)
        acc = acc + jnp.where(col == idxk, vk, jnp.int16(0))
        acc = acc + jnp.where(idxT[k:k + 1, :] == gid, vk, jnp.int16(0))
    out_ref[...] = acc.astype(jnp.float32) * 0.5


def _adj_build(vals8, idx, idxT):
    return pl.pallas_call(
        _adj_body,
        grid=(_NBLK,),
        in_specs=[
            pl.BlockSpec(memory_space=pltpu.SMEM),
            pl.BlockSpec((_RB, _K), lambda i: (i, 0)),
            pl.BlockSpec((_K, _N), lambda i: (0, 0)),
        ],
        out_specs=pl.BlockSpec((_RB, _N), lambda i: (i, 0)),
        out_shape=jax.ShapeDtypeStruct((_N, _N), jnp.float32),
    )(vals8, idx, idxT)


def kernel(image, n=8):
    features = _extract_patches(image)
    feat = features[0]
    valsi = (jnp.arange(_K) < n).astype(jnp.int32)
    idx = _tc_knn(feat)
    idxT16 = idx.T.astype(jnp.int16)
    adjacency = _adj_build(valsi, idx, idxT16)[None]
    return features, adjacency


# speedup vs baseline: 7.2773x; 1.3802x over previous
import jax
import jax.numpy as jnp
from jax import lax
from jax.experimental import pallas as pl
from jax.experimental.pallas import tpu as pltpu

_PH = _PW = _PD = 8
_N = 8192
_D = 512
_K = 8
_RB = 256
_NBLK = _N // _RB


def _feat_body(img_ref, out_ref):
    x = img_ref[0]
    x = x.reshape(_PH, 32, _PW, 8, _PD)
    x = jnp.transpose(x, (1, 3, 0, 2, 4))
    out_ref[...] = x.reshape(_RB, _D)


def _feat_extract(image):
    img = image.reshape(32, _PH, 256, 64)
    return pl.pallas_call(
        _feat_body,
        grid=(_NBLK,),
        in_specs=[pl.BlockSpec((1, _PH, 256, 64), lambda i: (i, 0, 0, 0))],
        out_specs=pl.BlockSpec((_RB, _D), lambda i: (i, 0)),
        out_shape=jax.ShapeDtypeStruct((_N, _D), jnp.float32),
    )(img)


def _tc_body(feat_ref, featA_ref, idx_ref, idxT_ref, sq_ref):
    i = pl.program_id(0)

    @pl.when(i == 0)
    def _():
        fa = featA_ref[...]
        sq_ref[...] = jnp.sum(fa * fa, axis=1)[None, :]

    score = sq_ref[...] - 2.0 * lax.dot_general(
        feat_ref[...], featA_ref[...],
        dimension_numbers=(((1,), (1,)), ((), ())),
        preferred_element_type=jnp.float32,
    )
    col = lax.broadcasted_iota(jnp.int32, (_RB, _N), 1).astype(jnp.float32)
    gid = (lax.broadcasted_iota(jnp.int32, (_RB, 1), 0) + i * _RB).astype(
        jnp.float32)
    score = jnp.where(col == gid, jnp.inf, score)

    for k in range(_K):
        m = jnp.min(score, axis=1, keepdims=True)
        a = jnp.min(jnp.where(score == m, col, jnp.float32(_N)),
                    axis=1, keepdims=True)
        score = jnp.where(col == a, jnp.inf, score)
        a16 = a.astype(jnp.int16)
        idx_ref[:, k:k + 1] = a16
        idxT_ref[k:k + 1, :] = a16.reshape(1, _RB)


def _tc_knn(feat):
    return pl.pallas_call(
        _tc_body,
        grid=(_NBLK,),
        in_specs=[
            pl.BlockSpec((_RB, _D), lambda i: (i, 0)),
            pl.BlockSpec((_N, _D), lambda i: (0, 0)),
        ],
        out_specs=[
            pl.BlockSpec((_RB, _K), lambda i: (i, 0)),
            pl.BlockSpec((_K, _RB), lambda i: (0, i)),
        ],
        out_shape=[
            jax.ShapeDtypeStruct((_N, _K), jnp.int16),
            jax.ShapeDtypeStruct((_K, _N), jnp.int16),
        ],
        scratch_shapes=[pltpu.VMEM((1, _N), jnp.float32)],
    )(feat, feat)


def _adj_body(vals_ref, idx_ref, idxT_ref, out_ref):
    i = pl.program_id(0)
    col = lax.broadcasted_iota(jnp.int16, (_RB, _N), 1)
    gid = lax.broadcasted_iota(jnp.int16, (_RB, 1), 0) + (i * _RB).astype(
        jnp.int16)
    idxT = idxT_ref[...]
    acc = jnp.zeros((_RB, _N), jnp.int16)
    for k in range(_K):
        vk = vals_ref[k].astype(jnp.int16)
        acc = acc + jnp.where(col == idx_ref[:, k:k + 1], vk, jnp.int16(0))
        acc = acc + jnp.where(idxT[k:k + 1, :] == gid, vk, jnp.int16(0))
    out_ref[...] = acc.astype(jnp.float32) * 0.5


def _adj_build(valsi, idx, idxT):
    return pl.pallas_call(
        _adj_body,
        grid=(_NBLK,),
        in_specs=[
            pl.BlockSpec(memory_space=pltpu.SMEM),
            pl.BlockSpec((_RB, _K), lambda i: (i, 0)),
            pl.BlockSpec((_K, _N), lambda i: (0, 0)),
        ],
        out_specs=pl.BlockSpec((_RB, _N), lambda i: (i, 0)),
        out_shape=jax.ShapeDtypeStruct((_N, _N), jnp.float32),
    )(valsi, idx, idxT)


def kernel(image, n=8):
    feat = _feat_extract(image)
    valsi = (jnp.arange(_K) < n).astype(jnp.int32)
    idx, idxT = _tc_knn(feat)
    adjacency = _adj_build(valsi, idx, idxT)[None]
    return feat.reshape(1, _N, _D), adjacency


# speedup vs baseline: 7.9439x; 1.0916x over previous
import jax
import jax.numpy as jnp
from jax import lax
from jax.experimental import pallas as pl
from jax.experimental.pallas import tpu as pltpu

_PH = _PW = _PD = 8
_N = 8192
_D = 512
_K = 8
_RB = 256
_NBLK = _N // _RB


def _feat_body(img_ref, out_ref):
    x = img_ref[0]
    x = x.reshape(_PH, 32, _PW, 8, _PD)
    x = jnp.transpose(x, (1, 3, 0, 2, 4))
    out_ref[...] = x.reshape(_RB, _D)


def _feat_extract(image):
    img = image.reshape(32, _PH, 256, 64)
    return pl.pallas_call(
        _feat_body,
        grid=(_NBLK,),
        in_specs=[pl.BlockSpec((1, _PH, 256, 64), lambda i: (i, 0, 0, 0))],
        out_specs=pl.BlockSpec((_RB, _D), lambda i: (i, 0)),
        out_shape=jax.ShapeDtypeStruct((_N, _D), jnp.float32),
    )(img)


def _tc_body(feat_ref, featA_ref, idx_ref, idxT_ref, sq_ref):
    i = pl.program_id(0)

    @pl.when(i == 0)
    def _():
        fa = featA_ref[...]
        sq_ref[...] = jnp.sum(fa * fa, axis=1)[None, :]

    score = sq_ref[...] - 2.0 * lax.dot_general(
        feat_ref[...], featA_ref[...],
        dimension_numbers=(((1,), (1,)), ((), ())),
        preferred_element_type=jnp.float32,
    )
    col = lax.broadcasted_iota(jnp.int32, (_RB, _N), 1).astype(jnp.float32)
    gid = (lax.broadcasted_iota(jnp.int32, (_RB, 1), 0) + i * _RB).astype(
        jnp.float32)
    score = jnp.where(col == gid, jnp.inf, score)

    for k in range(_K):
        m = jnp.min(score, axis=1, keepdims=True)
        a = jnp.min(jnp.where(score == m, col, jnp.float32(_N)),
                    axis=1, keepdims=True)
        score = jnp.where(col == a, jnp.inf, score)
        a16 = a.astype(jnp.int16)
        idx_ref[:, k:k + 1] = a16
        idxT_ref[k:k + 1, :] = a16.reshape(1, _RB)


def _tc_knn(feat):
    return pl.pallas_call(
        _tc_body,
        grid=(_NBLK,),
        in_specs=[
            pl.BlockSpec((_RB, _D), lambda i: (i, 0)),
            pl.BlockSpec((_N, _D), lambda i: (0, 0)),
        ],
        out_specs=[
            pl.BlockSpec((_RB, _K), lambda i: (i, 0)),
            pl.BlockSpec((_K, _RB), lambda i: (0, i)),
        ],
        out_shape=[
            jax.ShapeDtypeStruct((_N, _K), jnp.int16),
            jax.ShapeDtypeStruct((_K, _N), jnp.int16),
        ],
        scratch_shapes=[pltpu.VMEM((1, _N), jnp.float32)],
    )(feat, feat)


_CB = 2048
_NCBLK = _N // _CB


def _adj_body(vals_ref, idx_ref, idxT_ref, out_ref):
    i = pl.program_id(0)
    j = pl.program_id(1)
    col = lax.broadcasted_iota(jnp.int16, (_RB, _CB), 1) + (j * _CB).astype(
        jnp.int16)
    gid = lax.broadcasted_iota(jnp.int16, (_RB, 1), 0) + (i * _RB).astype(
        jnp.int16)
    idxT = idxT_ref[...]
    acc = jnp.zeros((_RB, _CB), jnp.int16)
    for k in range(_K):
        vk = vals_ref[k].astype(jnp.int16)
        acc = acc + jnp.where(col == idx_ref[:, k:k + 1], vk, jnp.int16(0))
        acc = acc + jnp.where(idxT[k:k + 1, :] == gid, vk, jnp.int16(0))
    out_ref[...] = acc.astype(jnp.float32) * 0.5


def _adj_build(valsi, idx, idxT):
    return pl.pallas_call(
        _adj_body,
        grid=(_NBLK, _NCBLK),
        in_specs=[
            pl.BlockSpec(memory_space=pltpu.SMEM),
            pl.BlockSpec((_RB, _K), lambda i, j: (i, 0)),
            pl.BlockSpec((_K, _CB), lambda i, j: (0, j)),
        ],
        out_specs=pl.BlockSpec((_RB, _CB), lambda i, j: (i, j)),
        out_shape=jax.ShapeDtypeStruct((_N, _N), jnp.float32),
    )(valsi, idx, idxT)


def kernel(image, n=8):
    feat = _feat_extract(image)
    valsi = (jnp.arange(_K) < n).astype(jnp.int32)
    idx, idxT = _tc_knn(feat)
    adjacency = _adj_build(valsi, idx, idxT)[None]
    return feat.reshape(1, _N, _D), adjacency
